# baseline (speedup 1.0000x reference)
import jax
import jax.numpy as jnp
from jax import lax
from jax.experimental import pallas as pl
from jax.experimental.pallas import tpu as pltpu
from jax.experimental.pallas import tpu_sc as plsc

_N = 10000
_E = 160000
_D = 256
_C = 40
_G = 128
_ITERS = 6
_EPS = 0.1
_GAMMA = 0.1

_NN = 10240
_NTILES = 16
_CHUNK = 128
_NCHUNK = 80
_EP = 2 * _NTILES * _NCHUNK * _CHUNK // 2
_RPT = _NN // _NTILES
_HD = _D // 2

_RB = 1024
_GRID = _NN // _RB

_sc_mesh = plsc.VectorSubcoreMesh(core_axis_name="c", subcore_axis_name="s")


def _sc_hist_body(dst_hbm, zo_hbm, o0_hbm, o1_hbm, idx_v, ob_v, acc_sh):
    c = lax.axis_index("c")
    s = lax.axis_index("s")
    w = c * _NTILES + s
    pltpu.sync_copy(zo_hbm.at[0], ob_v)

    @pl.loop(0, _RPT, step=_CHUNK)
    def _(r):
        pltpu.sync_copy(ob_v, acc_sh.at[pl.ds(s * _RPT + r, _CHUNK)])

    pltpu.sync_copy(dst_hbm.at[w], idx_v)
    pltpu.sync_copy(zo_hbm.at[1], ob_v)
    plsc.subcore_barrier()

    @pl.loop(0, 40)
    def _(j):
        pltpu.sync_copy(ob_v, acc_sh.at[idx_v.at[j]], add=True)

    plsc.subcore_barrier()
    row0 = s * _RPT

    @pl.when(c == 0)
    def _():
        pltpu.sync_copy(acc_sh.at[pl.ds(row0, _RPT)], o0_hbm.at[pl.ds(row0, _RPT)])

    @pl.when(c == 1)
    def _():
        pltpu.sync_copy(acc_sh.at[pl.ds(row0, _RPT)], o1_hbm.at[pl.ds(row0, _RPT)])


_hist_call = pl.kernel(
    _sc_hist_body,
    out_type=(jax.ShapeDtypeStruct((_NN, 16), jnp.float32),
              jax.ShapeDtypeStruct((_NN, 16), jnp.float32)),
    mesh=_sc_mesh,
    scratch_types=[
        pltpu.VMEM((40, _CHUNK), jnp.int32),
        pltpu.VMEM((_CHUNK, 16), jnp.float32),
        pltpu.VMEM_SHARED((_NN, 16), jnp.float32),
    ],
)


def _sc_edges_body(ua_hbm, ub_hbm, src_hbm, dst_hbm, z_hbm,
                   sa_hbm, sb_hbm, src_v, dst_v, ga_v, gb_v,
                   gsa, gsb, ssa, ssb, acc_sh):
    c = lax.axis_index("c")
    s = lax.axis_index("s")
    @pl.loop(0, _RPT, step=_CHUNK)
    def _(r):
        pltpu.async_copy(z_hbm, acc_sh.at[pl.ds(s * _RPT + r, _CHUNK)], ssa)

    _HC = _NCHUNK // 2

    def _edge_pass(u_hbm):
        for half in range(2):
            w = 2 * s + half
            pltpu.sync_copy(src_hbm.at[w], src_v)
            pltpu.sync_copy(dst_hbm.at[w], dst_v)
            pltpu.async_copy(u_hbm.at[src_v.at[0]], ga_v, gsa)
            pltpu.async_copy(u_hbm.at[src_v.at[1]], gb_v, gsb)
            if half == 0:
                @pl.loop(0, _RPT, step=_CHUNK)
                def _(r):
                    pltpu.make_async_copy(z_hbm, acc_sh.at[pl.ds(r, _CHUNK)],
                                          ssa).wait()
                plsc.subcore_barrier()

            @pl.loop(0, _HC, step=2)
            def _(j):
                pltpu.make_async_copy(u_hbm.at[src_v.at[j]], ga_v, gsa).wait()
                pltpu.async_copy(ga_v, acc_sh.at[dst_v.at[j]], ssa, add=True)
                pltpu.make_async_copy(u_hbm.at[src_v.at[j]], gb_v, gsb).wait()
                pltpu.async_copy(gb_v, acc_sh.at[dst_v.at[j + 1]], ssb, add=True)
                pltpu.make_async_copy(ga_v, acc_sh.at[dst_v.at[j]], ssa).wait()

                @pl.when(j + 2 < _HC)
                def _():
                    pltpu.async_copy(u_hbm.at[src_v.at[j + 2]], ga_v, gsa)

                pltpu.make_async_copy(gb_v, acc_sh.at[dst_v.at[j]], ssb).wait()

                @pl.when(j + 3 < _HC)
                def _():
                    pltpu.async_copy(u_hbm.at[src_v.at[j + 3]], gb_v, gsb)

    @pl.when(c == 0)
    def _():
        _edge_pass(ua_hbm)

    @pl.when(c == 1)
    def _():
        _edge_pass(ub_hbm)

    plsc.subcore_barrier()
    row0 = s * _RPT

    @pl.when(c == 0)
    def _():
        pltpu.sync_copy(acc_sh.at[pl.ds(row0, _RPT)], sa_hbm.at[pl.ds(row0, _RPT)])

    @pl.when(c == 1)
    def _():
        pltpu.sync_copy(acc_sh.at[pl.ds(row0, _RPT)], sb_hbm.at[pl.ds(row0, _RPT)])


_edges_call = pl.kernel(
    _sc_edges_body,
    out_type=(jax.ShapeDtypeStruct((_NN, _HD), jnp.float32),
              jax.ShapeDtypeStruct((_NN, _HD), jnp.float32)),
    mesh=_sc_mesh,
    scratch_types=[
        pltpu.VMEM((_NCHUNK // 2, _CHUNK), jnp.int32),
        pltpu.VMEM((_NCHUNK // 2, _CHUNK), jnp.int32),
        pltpu.VMEM((_CHUNK, _HD), jnp.float32),
        pltpu.VMEM((_CHUNK, _HD), jnp.float32),
        pltpu.SemaphoreType.DMA,
        pltpu.SemaphoreType.DMA,
        pltpu.SemaphoreType.DMA,
        pltpu.SemaphoreType.DMA,
        pltpu.VMEM_SHARED((_NN, _HD), jnp.float32),
    ],
)


def _mm(a, b):
    return lax.dot_general(a, b, (((1,), (0,)), ((), ())),
                           preferred_element_type=jnp.float32,
                           precision=lax.Precision.DEFAULT)


def _tc0_body(x_ref, m2_ref, h0_ref, h1_ref, d_ref, ua_ref, ub_ref):
    i = pl.program_id(0)
    deg = 1.0 + h0_ref[:, 0:1] + h1_ref[:, 0:1]
    rows = i * _RB + lax.broadcasted_iota(jnp.int32, (_RB, 1), 0)
    d = jnp.where(rows < _N, lax.rsqrt(deg), 0.0)
    d_ref[...] = d
    u = _mm(x_ref[...], m2_ref[...]) * d
    ua_ref[...] = u[:, :_HD]
    ub_ref[...] = u[:, _HD:]


def _tc0(x_p, m2, h0, h1):
    return pl.pallas_call(
        _tc0_body,
        grid=(_GRID,),
        in_specs=[
            pl.BlockSpec((_RB, _D), lambda i: (i, 0)),
            pl.BlockSpec((_D, _D), lambda i: (0, 0)),
            pl.BlockSpec((_RB, 16), lambda i: (i, 0)),
            pl.BlockSpec((_RB, 16), lambda i: (i, 0)),
        ],
        out_specs=[
            pl.BlockSpec((_RB, 1), lambda i: (i, 0)),
            pl.BlockSpec((_RB, _HD), lambda i: (i, 0)),
            pl.BlockSpec((_RB, _HD), lambda i: (i, 0)),
        ],
        out_shape=[
            jax.ShapeDtypeStruct((_NN, 1), jnp.float32),
            jax.ShapeDtypeStruct((_NN, _HD), jnp.float32),
            jax.ShapeDtypeStruct((_NN, _HD), jnp.float32),
        ],
    )(x_p, m2, h0, h1)


def _tci_body(h_ref, sa_ref, sb_ref, ua_ref, ub_ref, d_ref, m1_ref, m2_ref,
              b_ref, hn_ref, una_ref, unb_ref):
    h = h_ref[...]
    d = d_ref[...]
    su = jnp.concatenate([sa_ref[...] + ua_ref[...],
                          sb_ref[...] + ub_ref[...]], axis=1)
    z = _mm(h, m1_ref[...]) + d * su + b_ref[...]
    hn = h + _EPS * jnp.tanh(z)
    hn_ref[...] = hn
    un = _mm(hn, m2_ref[...]) * d
    una_ref[...] = un[:, :_HD]
    unb_ref[...] = un[:, _HD:]


def _tci(h, sa, sb, ua, ub, d, m1, m2, b2):
    return pl.pallas_call(
        _tci_body,
        grid=(_GRID,),
        in_specs=[
            pl.BlockSpec((_RB, _D), lambda i: (i, 0)),
            pl.BlockSpec((_RB, _HD), lambda i: (i, 0)),
            pl.BlockSpec((_RB, _HD), lambda i: (i, 0)),
            pl.BlockSpec((_RB, _HD), lambda i: (i, 0)),
            pl.BlockSpec((_RB, _HD), lambda i: (i, 0)),
            pl.BlockSpec((_RB, 1), lambda i: (i, 0)),
            pl.BlockSpec((_D, _D), lambda i: (0, 0)),
            pl.BlockSpec((_D, _D), lambda i: (0, 0)),
            pl.BlockSpec((1, _D), lambda i: (0, 0)),
        ],
        out_specs=[
            pl.BlockSpec((_RB, _D), lambda i: (i, 0)),
            pl.BlockSpec((_RB, _HD), lambda i: (i, 0)),
            pl.BlockSpec((_RB, _HD), lambda i: (i, 0)),
        ],
        out_shape=[
            jax.ShapeDtypeStruct((_NN, _D), jnp.float32),
            jax.ShapeDtypeStruct((_NN, _HD), jnp.float32),
            jax.ShapeDtypeStruct((_NN, _HD), jnp.float32),
        ],
    )(h, sa, sb, ua, ub, d, m1, m2, b2)


def _pool_body(h_ref, b_ref, fcw_ref, fcb_ref, o_ref, sums, counts):
    i = pl.program_id(0)

    @pl.when(i == 0)
    def _():
        sums[...] = jnp.zeros_like(sums)
        counts[...] = jnp.zeros_like(counts)

    seg = b_ref[...]
    gid = lax.broadcasted_iota(jnp.int32, (_RB, _G), 1)
    sel = (seg == gid).astype(jnp.float32)
    sums[...] += lax.dot_general(sel, h_ref[...], (((0,), (0,)), ((), ())),
                                 preferred_element_type=jnp.float32,
                                 precision=lax.Precision.HIGHEST)
    counts[...] += lax.dot_general(sel, jnp.ones((_RB, 1), jnp.float32),
                                   (((0,), (0,)), ((), ())),
                                   preferred_element_type=jnp.float32,
                                   precision=lax.Precision.HIGHEST)

    @pl.when(i == pl.num_programs(0) - 1)
    def _():
        pooled = sums[...] / jnp.maximum(counts[...], 1.0)
        o_ref[...] = _mm(pooled, fcw_ref[...]) + fcb_ref[...]


def _pool(h, batch_p, fcw, fcb):
    return pl.pallas_call(
        _pool_body,
        grid=(_GRID,),
        in_specs=[
            pl.BlockSpec((_RB, _D), lambda i: (i, 0)),
            pl.BlockSpec((_RB, 1), lambda i: (i, 0)),
            pl.BlockSpec((_D, _C), lambda i: (0, 0)),
            pl.BlockSpec((1, _C), lambda i: (0, 0)),
        ],
        out_specs=pl.BlockSpec((_G, _C), lambda i: (0, 0)),
        out_shape=jax.ShapeDtypeStruct((_G, _C), jnp.float32),
        scratch_shapes=[
            pltpu.VMEM((_G, _D), jnp.float32),
            pltpu.VMEM((_G, 1), jnp.float32),
        ],
    )(h, batch_p, fcw, fcb)


def kernel(x, edge_index, batch, W, bias, gcn_weight, fc_w, fc_b):
    f32 = jnp.float32
    npad = _NN - _N
    x_p = jnp.pad(x, ((0, npad), (0, 0)))
    epad = _EP - _E
    pad_idx = _N + (jnp.arange(epad, dtype=jnp.int32) % 64)
    order = jnp.argsort(edge_index[1])
    src_flat = jnp.concatenate([edge_index[0][order], pad_idx])
    dst_flat = jnp.concatenate([edge_index[1][order], pad_idx])
    src32 = src_flat.reshape(2 * _NTILES, _NCHUNK // 2, _CHUNK)
    dst32 = dst_flat.reshape(2 * _NTILES, _NCHUNK // 2, _CHUNK)
    batch_p = jnp.pad(batch, (0, npad), constant_values=_G).reshape(_NN, 1)

    m1 = (W - W.T - _GAMMA * jnp.eye(_D, dtype=f32)).T
    m2 = gcn_weight.T
    fcw = fc_w.T
    fcb = fc_b.reshape(1, _C)
    b2 = bias.reshape(1, _D)
    zo = jnp.stack([jnp.zeros((_CHUNK, 16), f32), jnp.ones((_CHUNK, 16), f32)])
    zrow = jnp.zeros((_CHUNK, _HD), f32)

    h0, h1 = _hist_call(dst32, zo)
    d, ua, ub = _tc0(x_p, m2, h0, h1)

    h = x_p
    for _ in range(_ITERS):
        sa, sb = _edges_call(ua, ub, src32, dst32, zrow)
        h, ua, ub = _tci(h, sa, sb, ua, ub, d, m1, m2, b2)

    return _pool(h, batch_p, fcw, fcb)


# speedup vs baseline: 1.2148x; 1.2148x over previous
import jax
import jax.numpy as jnp
from jax import lax
from jax.experimental import pallas as pl
from jax.experimental.pallas import tpu as pltpu
from jax.experimental.pallas import tpu_sc as plsc

_N = 10000
_E = 160000
_D = 256
_C = 40
_G = 128
_ITERS = 6
_EPS = 0.1
_GAMMA = 0.1

_NN = 10240
_NTILES = 16
_CHUNK = 128
_NCHUNK = 80
_EP = 2 * _NTILES * _NCHUNK * _CHUNK // 2
_RPT = _NN // _NTILES
_HD = _D // 2

_RB = 1024
_GRID = _NN // _RB

_sc_mesh = plsc.VectorSubcoreMesh(core_axis_name="c", subcore_axis_name="s")


def _sc_hist_body(dst_hbm, zo_hbm, o0_hbm, o1_hbm, idx_v, ob_v, acc_sh):
    c = lax.axis_index("c")
    s = lax.axis_index("s")
    w = c * _NTILES + s
    pltpu.sync_copy(zo_hbm.at[0], ob_v)

    @pl.loop(0, _RPT, step=_CHUNK)
    def _(r):
        pltpu.sync_copy(ob_v, acc_sh.at[pl.ds(s * _RPT + r, _CHUNK)])

    pltpu.sync_copy(dst_hbm.at[w], idx_v)
    pltpu.sync_copy(zo_hbm.at[1], ob_v)
    plsc.subcore_barrier()

    @pl.loop(0, 40)
    def _(j):
        pltpu.sync_copy(ob_v, acc_sh.at[idx_v.at[j]], add=True)

    plsc.subcore_barrier()
    row0 = s * _RPT

    @pl.when(c == 0)
    def _():
        pltpu.sync_copy(acc_sh.at[pl.ds(row0, _RPT)], o0_hbm.at[pl.ds(row0, _RPT)])

    @pl.when(c == 1)
    def _():
        pltpu.sync_copy(acc_sh.at[pl.ds(row0, _RPT)], o1_hbm.at[pl.ds(row0, _RPT)])


_hist_call = pl.kernel(
    _sc_hist_body,
    out_type=(jax.ShapeDtypeStruct((_NN, 16), jnp.float32),
              jax.ShapeDtypeStruct((_NN, 16), jnp.float32)),
    mesh=_sc_mesh,
    scratch_types=[
        pltpu.VMEM((40, _CHUNK), jnp.int32),
        pltpu.VMEM((_CHUNK, 16), jnp.float32),
        pltpu.VMEM_SHARED((_NN, 16), jnp.float32),
    ],
)


def _sc_edges_body(ua_hbm, ub_hbm, src_hbm, dst_hbm, z_hbm,
                   sa_hbm, sb_hbm, src_v, dst_v, ga_v, gb_v,
                   gsa, gsb, ssa, ssb, acc_sh):
    c = lax.axis_index("c")
    s = lax.axis_index("s")
    @pl.loop(0, _RPT, step=_CHUNK)
    def _(r):
        pltpu.async_copy(z_hbm, acc_sh.at[pl.ds(s * _RPT + r, _CHUNK)], ssa)

    _HC = _NCHUNK // 2

    def _edge_pass(u_hbm):
        for half in range(2):
            w = 2 * s + half
            pltpu.sync_copy(src_hbm.at[w], src_v)
            pltpu.sync_copy(dst_hbm.at[w], dst_v)
            pltpu.async_copy(u_hbm.at[src_v.at[0]], ga_v, gsa)
            pltpu.async_copy(u_hbm.at[src_v.at[1]], gb_v, gsb)
            if half == 0:
                @pl.loop(0, _RPT, step=_CHUNK)
                def _(r):
                    pltpu.make_async_copy(z_hbm, acc_sh.at[pl.ds(r, _CHUNK)],
                                          ssa).wait()
                plsc.subcore_barrier()

            @pl.loop(0, _HC, step=2)
            def _(j):
                pltpu.make_async_copy(u_hbm.at[src_v.at[j]], ga_v, gsa).wait()
                pltpu.async_copy(ga_v, acc_sh.at[dst_v.at[j]], ssa, add=True)
                pltpu.make_async_copy(u_hbm.at[src_v.at[j]], gb_v, gsb).wait()
                pltpu.async_copy(gb_v, acc_sh.at[dst_v.at[j + 1]], ssb, add=True)
                pltpu.make_async_copy(ga_v, acc_sh.at[dst_v.at[j]], ssa).wait()

                @pl.when(j + 2 < _HC)
                def _():
                    pltpu.async_copy(u_hbm.at[src_v.at[j + 2]], ga_v, gsa)

                pltpu.make_async_copy(gb_v, acc_sh.at[dst_v.at[j]], ssb).wait()

                @pl.when(j + 3 < _HC)
                def _():
                    pltpu.async_copy(u_hbm.at[src_v.at[j + 3]], gb_v, gsb)

    @pl.when(c == 0)
    def _():
        _edge_pass(ua_hbm)

    @pl.when(c == 1)
    def _():
        _edge_pass(ub_hbm)

    plsc.subcore_barrier()
    row0 = s * _RPT

    @pl.when(c == 0)
    def _():
        pltpu.sync_copy(acc_sh.at[pl.ds(row0, _RPT)], sa_hbm.at[pl.ds(row0, _RPT)])

    @pl.when(c == 1)
    def _():
        pltpu.sync_copy(acc_sh.at[pl.ds(row0, _RPT)], sb_hbm.at[pl.ds(row0, _RPT)])


_edges_call = pl.kernel(
    _sc_edges_body,
    out_type=(jax.ShapeDtypeStruct((_NN, _HD), jnp.float32),
              jax.ShapeDtypeStruct((_NN, _HD), jnp.float32)),
    mesh=_sc_mesh,
    scratch_types=[
        pltpu.VMEM((_NCHUNK // 2, _CHUNK), jnp.int32),
        pltpu.VMEM((_NCHUNK // 2, _CHUNK), jnp.int32),
        pltpu.VMEM((_CHUNK, _HD), jnp.float32),
        pltpu.VMEM((_CHUNK, _HD), jnp.float32),
        pltpu.SemaphoreType.DMA,
        pltpu.SemaphoreType.DMA,
        pltpu.SemaphoreType.DMA,
        pltpu.SemaphoreType.DMA,
        pltpu.VMEM_SHARED((_NN, _HD), jnp.float32),
    ],
)


def _mm(a, b):
    return lax.dot_general(a, b, (((1,), (0,)), ((), ())),
                           preferred_element_type=jnp.float32,
                           precision=lax.Precision.DEFAULT)


def _tc0_body(x_ref, m2_ref, h0_ref, h1_ref, d_ref, ua_ref, ub_ref):
    i = pl.program_id(0)
    deg = 1.0 + h0_ref[:, 0:1] + h1_ref[:, 0:1]
    rows = i * _RB + lax.broadcasted_iota(jnp.int32, (_RB, 1), 0)
    d = jnp.where(rows < _N, lax.rsqrt(deg), 0.0)
    d_ref[...] = d
    u = _mm(x_ref[...], m2_ref[...]) * d
    ua_ref[...] = u[:, :_HD]
    ub_ref[...] = u[:, _HD:]


def _tc0(x_p, m2, h0, h1):
    return pl.pallas_call(
        _tc0_body,
        grid=(_GRID,),
        in_specs=[
            pl.BlockSpec((_RB, _D), lambda i: (i, 0)),
            pl.BlockSpec((_D, _D), lambda i: (0, 0)),
            pl.BlockSpec((_RB, 16), lambda i: (i, 0)),
            pl.BlockSpec((_RB, 16), lambda i: (i, 0)),
        ],
        out_specs=[
            pl.BlockSpec((_RB, 1), lambda i: (i, 0)),
            pl.BlockSpec((_RB, _HD), lambda i: (i, 0)),
            pl.BlockSpec((_RB, _HD), lambda i: (i, 0)),
        ],
        out_shape=[
            jax.ShapeDtypeStruct((_NN, 1), jnp.float32),
            jax.ShapeDtypeStruct((_NN, _HD), jnp.float32),
            jax.ShapeDtypeStruct((_NN, _HD), jnp.float32),
        ],
    )(x_p, m2, h0, h1)


def _tci_body(h_ref, sa_ref, sb_ref, ua_ref, ub_ref, d_ref, m1_ref, m2_ref,
              b_ref, hn_ref, una_ref, unb_ref):
    h = h_ref[...]
    d = d_ref[...]
    su = jnp.concatenate([sa_ref[...] + ua_ref[...],
                          sb_ref[...] + ub_ref[...]], axis=1)
    z = _mm(h, m1_ref[...]) + d * su + b_ref[...]
    hn = h + _EPS * jnp.tanh(z)
    hn_ref[...] = hn
    un = _mm(hn, m2_ref[...]) * d
    una_ref[...] = un[:, :_HD]
    unb_ref[...] = un[:, _HD:]


def _tci(h, sa, sb, ua, ub, d, m1, m2, b2):
    return pl.pallas_call(
        _tci_body,
        grid=(_GRID,),
        in_specs=[
            pl.BlockSpec((_RB, _D), lambda i: (i, 0)),
            pl.BlockSpec((_RB, _HD), lambda i: (i, 0)),
            pl.BlockSpec((_RB, _HD), lambda i: (i, 0)),
            pl.BlockSpec((_RB, _HD), lambda i: (i, 0)),
            pl.BlockSpec((_RB, _HD), lambda i: (i, 0)),
            pl.BlockSpec((_RB, 1), lambda i: (i, 0)),
            pl.BlockSpec((_D, _D), lambda i: (0, 0)),
            pl.BlockSpec((_D, _D), lambda i: (0, 0)),
            pl.BlockSpec((1, _D), lambda i: (0, 0)),
        ],
        out_specs=[
            pl.BlockSpec((_RB, _D), lambda i: (i, 0)),
            pl.BlockSpec((_RB, _HD), lambda i: (i, 0)),
            pl.BlockSpec((_RB, _HD), lambda i: (i, 0)),
        ],
        out_shape=[
            jax.ShapeDtypeStruct((_NN, _D), jnp.float32),
            jax.ShapeDtypeStruct((_NN, _HD), jnp.float32),
            jax.ShapeDtypeStruct((_NN, _HD), jnp.float32),
        ],
    )(h, sa, sb, ua, ub, d, m1, m2, b2)


def _pool_body(h_ref, b_ref, fcw_ref, fcb_ref, o_ref, sums, counts):
    i = pl.program_id(0)

    @pl.when(i == 0)
    def _():
        sums[...] = jnp.zeros_like(sums)
        counts[...] = jnp.zeros_like(counts)

    seg = b_ref[...]
    gid = lax.broadcasted_iota(jnp.int32, (_RB, _G), 1)
    sel = (seg == gid).astype(jnp.float32)
    sums[...] += lax.dot_general(sel, h_ref[...], (((0,), (0,)), ((), ())),
                                 preferred_element_type=jnp.float32,
                                 precision=lax.Precision.HIGHEST)
    counts[...] += lax.dot_general(sel, jnp.ones((_RB, 1), jnp.float32),
                                   (((0,), (0,)), ((), ())),
                                   preferred_element_type=jnp.float32,
                                   precision=lax.Precision.HIGHEST)

    @pl.when(i == pl.num_programs(0) - 1)
    def _():
        pooled = sums[...] / jnp.maximum(counts[...], 1.0)
        o_ref[...] = _mm(pooled, fcw_ref[...]) + fcb_ref[...]


def _pool(h, batch_p, fcw, fcb):
    return pl.pallas_call(
        _pool_body,
        grid=(_GRID,),
        in_specs=[
            pl.BlockSpec((_RB, _D), lambda i: (i, 0)),
            pl.BlockSpec((_RB, 1), lambda i: (i, 0)),
            pl.BlockSpec((_D, _C), lambda i: (0, 0)),
            pl.BlockSpec((1, _C), lambda i: (0, 0)),
        ],
        out_specs=pl.BlockSpec((_G, _C), lambda i: (0, 0)),
        out_shape=jax.ShapeDtypeStruct((_G, _C), jnp.float32),
        scratch_shapes=[
            pltpu.VMEM((_G, _D), jnp.float32),
            pltpu.VMEM((_G, 1), jnp.float32),
        ],
    )(h, batch_p, fcw, fcb)


def kernel(x, edge_index, batch, W, bias, gcn_weight, fc_w, fc_b):
    f32 = jnp.float32
    npad = _NN - _N
    x_p = jnp.pad(x, ((0, npad), (0, 0)))
    epad = _EP - _E
    pad_idx = _N + (jnp.arange(epad, dtype=jnp.int32) % 64)
    src_flat = jnp.concatenate([edge_index[0], pad_idx])
    dst_flat = jnp.concatenate([edge_index[1], pad_idx])
    src32 = src_flat.reshape(2 * _NTILES, _NCHUNK // 2, _CHUNK)
    dst32 = dst_flat.reshape(2 * _NTILES, _NCHUNK // 2, _CHUNK)
    batch_p = jnp.pad(batch, (0, npad), constant_values=_G).reshape(_NN, 1)

    m1 = (W - W.T - _GAMMA * jnp.eye(_D, dtype=f32)).T
    m2 = gcn_weight.T
    fcw = fc_w.T
    fcb = fc_b.reshape(1, _C)
    b2 = bias.reshape(1, _D)
    zo = jnp.stack([jnp.zeros((_CHUNK, 16), f32), jnp.ones((_CHUNK, 16), f32)])
    zrow = jnp.zeros((_CHUNK, _HD), f32)

    h0, h1 = _hist_call(dst32, zo)
    d, ua, ub = _tc0(x_p, m2, h0, h1)

    h = x_p
    for _ in range(_ITERS):
        sa, sb = _edges_call(ua, ub, src32, dst32, zrow)
        h, ua, ub = _tci(h, sa, sb, ua, ub, d, m1, m2, b2)

    return _pool(h, batch_p, fcw, fcb)


# speedup vs baseline: 1.2368x; 1.0181x over previous
import jax
import jax.numpy as jnp
from jax import lax
from jax.experimental import pallas as pl
from jax.experimental.pallas import tpu as pltpu
from jax.experimental.pallas import tpu_sc as plsc

_N = 10000
_E = 160000
_D = 256
_C = 40
_G = 128
_ITERS = 6
_EPS = 0.1
_GAMMA = 0.1

_NN = 10240
_NTILES = 16
_CHUNK = 128
_NCHUNK = 80
_EP = 2 * _NTILES * _NCHUNK * _CHUNK // 2
_RPT = _NN // _NTILES
_HD = _D // 2

_RB = 1024
_GRID = _NN // _RB

_sc_mesh = plsc.VectorSubcoreMesh(core_axis_name="c", subcore_axis_name="s")


def _sc_hist_body(dst_hbm, zo_hbm, o0_hbm, o1_hbm, idx_v, ob_v, acc_sh):
    c = lax.axis_index("c")
    s = lax.axis_index("s")
    w = c * _NTILES + s
    pltpu.sync_copy(zo_hbm.at[0], ob_v)

    @pl.loop(0, _RPT, step=_CHUNK)
    def _(r):
        pltpu.sync_copy(ob_v, acc_sh.at[pl.ds(s * _RPT + r, _CHUNK)])

    pltpu.sync_copy(dst_hbm.at[w], idx_v)
    pltpu.sync_copy(zo_hbm.at[1], ob_v)
    plsc.subcore_barrier()

    @pl.loop(0, 40)
    def _(j):
        pltpu.sync_copy(ob_v, acc_sh.at[idx_v.at[j]], add=True)

    plsc.subcore_barrier()
    row0 = s * _RPT

    @pl.when(c == 0)
    def _():
        pltpu.sync_copy(acc_sh.at[pl.ds(row0, _RPT)], o0_hbm.at[pl.ds(row0, _RPT)])

    @pl.when(c == 1)
    def _():
        pltpu.sync_copy(acc_sh.at[pl.ds(row0, _RPT)], o1_hbm.at[pl.ds(row0, _RPT)])


_hist_call = pl.kernel(
    _sc_hist_body,
    out_type=(jax.ShapeDtypeStruct((_NN, 16), jnp.float32),
              jax.ShapeDtypeStruct((_NN, 16), jnp.float32)),
    mesh=_sc_mesh,
    scratch_types=[
        pltpu.VMEM((40, _CHUNK), jnp.int32),
        pltpu.VMEM((_CHUNK, 16), jnp.float32),
        pltpu.VMEM_SHARED((_NN, 16), jnp.float32),
    ],
)


def _sc_edges_body(ua_hbm, ub_hbm, src_hbm, dst_hbm, z_hbm,
                   sa_hbm, sb_hbm, src_v, dst_v, ga_v, gb_v,
                   gsa, gsb, ssa, ssb, acc_sh):
    c = lax.axis_index("c")
    s = lax.axis_index("s")
    pltpu.sync_copy(z_hbm, ga_v)

    @pl.loop(0, _RPT, step=_CHUNK)
    def _(r):
        pltpu.sync_copy(ga_v, acc_sh.at[pl.ds(s * _RPT + r, _CHUNK)])

    plsc.subcore_barrier()
    _HC = _NCHUNK // 2

    def _edge_pass(u_hbm):
        for half in range(2):
            w = 2 * s + half
            pltpu.sync_copy(src_hbm.at[w], src_v)
            pltpu.sync_copy(dst_hbm.at[w], dst_v)
            pltpu.async_copy(u_hbm.at[src_v.at[0]], ga_v, gsa)
            pltpu.async_copy(u_hbm.at[src_v.at[1]], gb_v, gsb)

            @pl.loop(0, _HC, step=2)
            def _(j):
                pltpu.make_async_copy(u_hbm.at[src_v.at[j]], ga_v, gsa).wait()
                pltpu.async_copy(ga_v, acc_sh.at[dst_v.at[j]], ssa, add=True)
                pltpu.make_async_copy(u_hbm.at[src_v.at[j]], gb_v, gsb).wait()
                pltpu.async_copy(gb_v, acc_sh.at[dst_v.at[j + 1]], ssb, add=True)
                pltpu.make_async_copy(ga_v, acc_sh.at[dst_v.at[j]], ssa).wait()

                @pl.when(j + 2 < _HC)
                def _():
                    pltpu.async_copy(u_hbm.at[src_v.at[j + 2]], ga_v, gsa)

                pltpu.make_async_copy(gb_v, acc_sh.at[dst_v.at[j]], ssb).wait()

                @pl.when(j + 3 < _HC)
                def _():
                    pltpu.async_copy(u_hbm.at[src_v.at[j + 3]], gb_v, gsb)

    @pl.when(c == 0)
    def _():
        _edge_pass(ua_hbm)

    @pl.when(c == 1)
    def _():
        _edge_pass(ub_hbm)

    plsc.subcore_barrier()
    row0 = s * _RPT

    @pl.when(c == 0)
    def _():
        pltpu.sync_copy(acc_sh.at[pl.ds(row0, _RPT)], sa_hbm.at[pl.ds(row0, _RPT)])

    @pl.when(c == 1)
    def _():
        pltpu.sync_copy(acc_sh.at[pl.ds(row0, _RPT)], sb_hbm.at[pl.ds(row0, _RPT)])


_edges_call = pl.kernel(
    _sc_edges_body,
    out_type=(jax.ShapeDtypeStruct((_NN, _HD), jnp.float32),
              jax.ShapeDtypeStruct((_NN, _HD), jnp.float32)),
    mesh=_sc_mesh,
    scratch_types=[
        pltpu.VMEM((_NCHUNK // 2, _CHUNK), jnp.int32),
        pltpu.VMEM((_NCHUNK // 2, _CHUNK), jnp.int32),
        pltpu.VMEM((_CHUNK, _HD), jnp.float32),
        pltpu.VMEM((_CHUNK, _HD), jnp.float32),
        pltpu.SemaphoreType.DMA,
        pltpu.SemaphoreType.DMA,
        pltpu.SemaphoreType.DMA,
        pltpu.SemaphoreType.DMA,
        pltpu.VMEM_SHARED((_NN, _HD), jnp.float32),
    ],
)


def _mm(a, b):
    return lax.dot_general(a, b, (((1,), (0,)), ((), ())),
                           preferred_element_type=jnp.float32,
                           precision=lax.Precision.DEFAULT)


def _tc0_body(x_ref, m2_ref, h0_ref, h1_ref, d_ref, ua_ref, ub_ref):
    i = pl.program_id(0)
    deg = 1.0 + h0_ref[:, 0:1] + h1_ref[:, 0:1]
    rows = i * _RB + lax.broadcasted_iota(jnp.int32, (_RB, 1), 0)
    d = jnp.where(rows < _N, lax.rsqrt(deg), 0.0)
    d_ref[...] = d
    u = _mm(x_ref[...], m2_ref[...]) * d
    ua_ref[...] = u[:, :_HD]
    ub_ref[...] = u[:, _HD:]


def _tc0(x_p, m2, h0, h1):
    return pl.pallas_call(
        _tc0_body,
        grid=(_GRID,),
        in_specs=[
            pl.BlockSpec((_RB, _D), lambda i: (i, 0)),
            pl.BlockSpec((_D, _D), lambda i: (0, 0)),
            pl.BlockSpec((_RB, 16), lambda i: (i, 0)),
            pl.BlockSpec((_RB, 16), lambda i: (i, 0)),
        ],
        out_specs=[
            pl.BlockSpec((_RB, 1), lambda i: (i, 0)),
            pl.BlockSpec((_RB, _HD), lambda i: (i, 0)),
            pl.BlockSpec((_RB, _HD), lambda i: (i, 0)),
        ],
        out_shape=[
            jax.ShapeDtypeStruct((_NN, 1), jnp.float32),
            jax.ShapeDtypeStruct((_NN, _HD), jnp.float32),
            jax.ShapeDtypeStruct((_NN, _HD), jnp.float32),
        ],
    )(x_p, m2, h0, h1)


def _tcz_body(h_ref, m1_ref, b_ref, p_ref):
    p_ref[...] = _mm(h_ref[...], m1_ref[...]) + b_ref[...]


def _tcz(h, m1, b2):
    return pl.pallas_call(
        _tcz_body,
        grid=(_GRID,),
        in_specs=[
            pl.BlockSpec((_RB, _D), lambda i: (i, 0)),
            pl.BlockSpec((_D, _D), lambda i: (0, 0)),
            pl.BlockSpec((1, _D), lambda i: (0, 0)),
        ],
        out_specs=pl.BlockSpec((_RB, _D), lambda i: (i, 0)),
        out_shape=jax.ShapeDtypeStruct((_NN, _D), jnp.float32),
    )(h, m1, b2)


def _tci_body(h_ref, p_ref, sa_ref, sb_ref, ua_ref, ub_ref, d_ref, m2_ref,
              hn_ref, una_ref, unb_ref):
    h = h_ref[...]
    d = d_ref[...]
    su = jnp.concatenate([sa_ref[...] + ua_ref[...],
                          sb_ref[...] + ub_ref[...]], axis=1)
    z = p_ref[...] + d * su
    hn = h + _EPS * jnp.tanh(z)
    hn_ref[...] = hn
    un = _mm(hn, m2_ref[...]) * d
    una_ref[...] = un[:, :_HD]
    unb_ref[...] = un[:, _HD:]


def _tci(h, p, sa, sb, ua, ub, d, m2):
    return pl.pallas_call(
        _tci_body,
        grid=(_GRID,),
        in_specs=[
            pl.BlockSpec((_RB, _D), lambda i: (i, 0)),
            pl.BlockSpec((_RB, _D), lambda i: (i, 0)),
            pl.BlockSpec((_RB, _HD), lambda i: (i, 0)),
            pl.BlockSpec((_RB, _HD), lambda i: (i, 0)),
            pl.BlockSpec((_RB, _HD), lambda i: (i, 0)),
            pl.BlockSpec((_RB, _HD), lambda i: (i, 0)),
            pl.BlockSpec((_RB, 1), lambda i: (i, 0)),
            pl.BlockSpec((_D, _D), lambda i: (0, 0)),
        ],
        out_specs=[
            pl.BlockSpec((_RB, _D), lambda i: (i, 0)),
            pl.BlockSpec((_RB, _HD), lambda i: (i, 0)),
            pl.BlockSpec((_RB, _HD), lambda i: (i, 0)),
        ],
        out_shape=[
            jax.ShapeDtypeStruct((_NN, _D), jnp.float32),
            jax.ShapeDtypeStruct((_NN, _HD), jnp.float32),
            jax.ShapeDtypeStruct((_NN, _HD), jnp.float32),
        ],
    )(h, p, sa, sb, ua, ub, d, m2)


def _pool_body(h_ref, b_ref, fcw_ref, fcb_ref, o_ref, sums, counts):
    i = pl.program_id(0)

    @pl.when(i == 0)
    def _():
        sums[...] = jnp.zeros_like(sums)
        counts[...] = jnp.zeros_like(counts)

    seg = b_ref[...]
    gid = lax.broadcasted_iota(jnp.int32, (_RB, _G), 1)
    sel = (seg == gid).astype(jnp.float32)
    sums[...] += lax.dot_general(sel, h_ref[...], (((0,), (0,)), ((), ())),
                                 preferred_element_type=jnp.float32,
                                 precision=lax.Precision.HIGHEST)
    counts[...] += lax.dot_general(sel, jnp.ones((_RB, 1), jnp.float32),
                                   (((0,), (0,)), ((), ())),
                                   preferred_element_type=jnp.float32,
                                   precision=lax.Precision.HIGHEST)

    @pl.when(i == pl.num_programs(0) - 1)
    def _():
        pooled = sums[...] / jnp.maximum(counts[...], 1.0)
        o_ref[...] = _mm(pooled, fcw_ref[...]) + fcb_ref[...]


def _pool(h, batch_p, fcw, fcb):
    return pl.pallas_call(
        _pool_body,
        grid=(_GRID,),
        in_specs=[
            pl.BlockSpec((_RB, _D), lambda i: (i, 0)),
            pl.BlockSpec((_RB, 1), lambda i: (i, 0)),
            pl.BlockSpec((_D, _C), lambda i: (0, 0)),
            pl.BlockSpec((1, _C), lambda i: (0, 0)),
        ],
        out_specs=pl.BlockSpec((_G, _C), lambda i: (0, 0)),
        out_shape=jax.ShapeDtypeStruct((_G, _C), jnp.float32),
        scratch_shapes=[
            pltpu.VMEM((_G, _D), jnp.float32),
            pltpu.VMEM((_G, 1), jnp.float32),
        ],
    )(h, batch_p, fcw, fcb)


def kernel(x, edge_index, batch, W, bias, gcn_weight, fc_w, fc_b):
    f32 = jnp.float32
    npad = _NN - _N
    x_p = jnp.pad(x, ((0, npad), (0, 0)))
    epad = _EP - _E
    pad_idx = _N + (jnp.arange(epad, dtype=jnp.int32) % 64)
    src_flat = jnp.concatenate([edge_index[0], pad_idx])
    dst_flat = jnp.concatenate([edge_index[1], pad_idx])
    src32 = src_flat.reshape(2 * _NTILES, _NCHUNK // 2, _CHUNK)
    dst32 = dst_flat.reshape(2 * _NTILES, _NCHUNK // 2, _CHUNK)
    batch_p = jnp.pad(batch, (0, npad), constant_values=_G).reshape(_NN, 1)

    m1 = (W - W.T - _GAMMA * jnp.eye(_D, dtype=f32)).T
    m2 = gcn_weight.T
    fcw = fc_w.T
    fcb = fc_b.reshape(1, _C)
    b2 = bias.reshape(1, _D)
    zo = jnp.stack([jnp.zeros((_CHUNK, 16), f32), jnp.ones((_CHUNK, 16), f32)])
    zrow = jnp.zeros((_CHUNK, _HD), f32)

    h0, h1 = _hist_call(dst32, zo)
    d, ua, ub = _tc0(x_p, m2, h0, h1)

    h = x_p
    for _ in range(_ITERS):
        p = _tcz(h, m1, b2)
        sa, sb = _edges_call(ua, ub, src32, dst32, zrow)
        h, ua, ub = _tci(h, p, sa, sb, ua, ub, d, m2)

    return _pool(h, batch_p, fcw, fcb)


# speedup vs baseline: 1.4900x; 1.2048x over previous
import jax
import jax.numpy as jnp
from jax import lax
from jax.experimental import pallas as pl
from jax.experimental.pallas import tpu as pltpu
from jax.experimental.pallas import tpu_sc as plsc

_N = 10000
_E = 160000
_D = 256
_C = 40
_G = 128
_ITERS = 6
_EPS = 0.1
_GAMMA = 0.1

_NN = 10240
_NTILES = 16
_CHUNK = 80
_NPART = 64
_CPP = 32
_EP = _NPART * _CPP * _CHUNK
_RPT = _NN // _NTILES
_HD = _D // 2

_RB = 1024
_GRID = _NN // _RB

_sc_mesh = plsc.VectorSubcoreMesh(core_axis_name="c", subcore_axis_name="s")


def _sc_hist_body(dst_hbm, zo_hbm, o0_hbm, o1_hbm, idx_v, ob_v, acc_sh):
    c = lax.axis_index("c")
    s = lax.axis_index("s")
    w = c * _NTILES + s
    pltpu.sync_copy(zo_hbm.at[0], ob_v)

    @pl.loop(0, _RPT, step=128)
    def _(r):
        pltpu.sync_copy(ob_v, acc_sh.at[pl.ds(s * _RPT + r, 128)])

    pltpu.sync_copy(dst_hbm.at[w], idx_v)
    pltpu.sync_copy(zo_hbm.at[1], ob_v)
    plsc.subcore_barrier()

    @pl.loop(0, 40)
    def _(j):
        pltpu.sync_copy(ob_v, acc_sh.at[idx_v.at[j]], add=True)

    plsc.subcore_barrier()
    row0 = s * _RPT

    @pl.when(c == 0)
    def _():
        pltpu.sync_copy(acc_sh.at[pl.ds(row0, _RPT)], o0_hbm.at[pl.ds(row0, _RPT)])

    @pl.when(c == 1)
    def _():
        pltpu.sync_copy(acc_sh.at[pl.ds(row0, _RPT)], o1_hbm.at[pl.ds(row0, _RPT)])


_hist_call = pl.kernel(
    _sc_hist_body,
    out_type=(jax.ShapeDtypeStruct((_NN, 16), jnp.float32),
              jax.ShapeDtypeStruct((_NN, 16), jnp.float32)),
    mesh=_sc_mesh,
    scratch_types=[
        pltpu.VMEM((40, 128), jnp.int32),
        pltpu.VMEM((128, 16), jnp.float32),
        pltpu.VMEM_SHARED((_NN, 16), jnp.float32),
    ],
)


def _sc_edges_body(ua_hbm, ub_hbm, src_hbm, dst_hbm, z_hbm,
                   sa_hbm, sb_hbm, src_v, dst_v,
                   g0, g1, g2, g3, gs0, gs1, gs2, gs3,
                   ss0, ss1, ss2, ss3, acc_sh):
    c = lax.axis_index("c")
    s = lax.axis_index("s")
    bufs = [g0, g1, g2, g3]
    gss = [gs0, gs1, gs2, gs3]
    sss = [ss0, ss1, ss2, ss3]
    pltpu.sync_copy(z_hbm, g0)

    @pl.loop(0, _RPT, step=_CHUNK)
    def _(r):
        pltpu.sync_copy(g0, acc_sh.at[pl.ds(s * _RPT + r, _CHUNK)])

    plsc.subcore_barrier()

    def _edge_pass(u_hbm):
        for part in range(4):
            w = 4 * s + part
            pltpu.sync_copy(src_hbm.at[w], src_v)
            pltpu.sync_copy(dst_hbm.at[w], dst_v)
            for k in range(4):
                pltpu.async_copy(u_hbm.at[src_v.at[k]], bufs[k], gss[k])

            @pl.loop(0, _CPP, step=4)
            def _(j):
                for k in range(4):
                    pltpu.make_async_copy(u_hbm.at[src_v.at[j]],
                                          bufs[k], gss[k]).wait()
                    pltpu.async_copy(bufs[k], acc_sh.at[dst_v.at[j + k]],
                                     sss[k], add=True)
                for k in range(4):
                    pltpu.make_async_copy(bufs[k], acc_sh.at[dst_v.at[j]],
                                          sss[k]).wait()

                    @pl.when(j + 4 + k < _CPP)
                    def _():
                        pltpu.async_copy(u_hbm.at[src_v.at[j + 4 + k]],
                                         bufs[k], gss[k])

    @pl.when(c == 0)
    def _():
        _edge_pass(ua_hbm)

    @pl.when(c == 1)
    def _():
        _edge_pass(ub_hbm)

    plsc.subcore_barrier()
    row0 = s * _RPT

    @pl.when(c == 0)
    def _():
        pltpu.sync_copy(acc_sh.at[pl.ds(row0, _RPT)], sa_hbm.at[pl.ds(row0, _RPT)])

    @pl.when(c == 1)
    def _():
        pltpu.sync_copy(acc_sh.at[pl.ds(row0, _RPT)], sb_hbm.at[pl.ds(row0, _RPT)])


_edges_call = pl.kernel(
    _sc_edges_body,
    out_type=(jax.ShapeDtypeStruct((_NN, _HD), jnp.float32),
              jax.ShapeDtypeStruct((_NN, _HD), jnp.float32)),
    mesh=_sc_mesh,
    scratch_types=(
        [pltpu.VMEM((_CPP, _CHUNK), jnp.int32),
         pltpu.VMEM((_CPP, _CHUNK), jnp.int32)]
        + [pltpu.VMEM((_CHUNK, _HD), jnp.float32)] * 4
        + [pltpu.SemaphoreType.DMA] * 8
        + [pltpu.VMEM_SHARED((_NN, _HD), jnp.float32)]
    ),
)


def _mm(a, b):
    return lax.dot_general(a, b, (((1,), (0,)), ((), ())),
                           preferred_element_type=jnp.float32,
                           precision=lax.Precision.DEFAULT)


def _tc0_body(x_ref, m2_ref, h0_ref, h1_ref, d_ref, ua_ref, ub_ref):
    i = pl.program_id(0)
    deg = 1.0 + h0_ref[:, 0:1] + h1_ref[:, 0:1]
    rows = i * _RB + lax.broadcasted_iota(jnp.int32, (_RB, 1), 0)
    d = jnp.where(rows < _N, lax.rsqrt(deg), 0.0)
    d_ref[...] = d
    u = _mm(x_ref[...], m2_ref[...]) * d
    ua_ref[...] = u[:, :_HD]
    ub_ref[...] = u[:, _HD:]


def _tc0(x_p, m2, h0, h1):
    return pl.pallas_call(
        _tc0_body,
        grid=(_GRID,),
        in_specs=[
            pl.BlockSpec((_RB, _D), lambda i: (i, 0)),
            pl.BlockSpec((_D, _D), lambda i: (0, 0)),
            pl.BlockSpec((_RB, 16), lambda i: (i, 0)),
            pl.BlockSpec((_RB, 16), lambda i: (i, 0)),
        ],
        out_specs=[
            pl.BlockSpec((_RB, 1), lambda i: (i, 0)),
            pl.BlockSpec((_RB, _HD), lambda i: (i, 0)),
            pl.BlockSpec((_RB, _HD), lambda i: (i, 0)),
        ],
        out_shape=[
            jax.ShapeDtypeStruct((_NN, 1), jnp.float32),
            jax.ShapeDtypeStruct((_NN, _HD), jnp.float32),
            jax.ShapeDtypeStruct((_NN, _HD), jnp.float32),
        ],
    )(x_p, m2, h0, h1)


def _tci_body(h_ref, sa_ref, sb_ref, ua_ref, ub_ref, d_ref, m1_ref, m2_ref,
              b_ref, hn_ref, una_ref, unb_ref):
    h = h_ref[...]
    d = d_ref[...]
    su = jnp.concatenate([sa_ref[...] + ua_ref[...],
                          sb_ref[...] + ub_ref[...]], axis=1)
    z = _mm(h, m1_ref[...]) + d * su + b_ref[...]
    hn = h + _EPS * jnp.tanh(z)
    hn_ref[...] = hn
    un = _mm(hn, m2_ref[...]) * d
    una_ref[...] = un[:, :_HD]
    unb_ref[...] = un[:, _HD:]


def _tci(h, sa, sb, ua, ub, d, m1, m2, b2):
    return pl.pallas_call(
        _tci_body,
        grid=(_GRID,),
        in_specs=[
            pl.BlockSpec((_RB, _D), lambda i: (i, 0)),
            pl.BlockSpec((_RB, _HD), lambda i: (i, 0)),
            pl.BlockSpec((_RB, _HD), lambda i: (i, 0)),
            pl.BlockSpec((_RB, _HD), lambda i: (i, 0)),
            pl.BlockSpec((_RB, _HD), lambda i: (i, 0)),
            pl.BlockSpec((_RB, 1), lambda i: (i, 0)),
            pl.BlockSpec((_D, _D), lambda i: (0, 0)),
            pl.BlockSpec((_D, _D), lambda i: (0, 0)),
            pl.BlockSpec((1, _D), lambda i: (0, 0)),
        ],
        out_specs=[
            pl.BlockSpec((_RB, _D), lambda i: (i, 0)),
            pl.BlockSpec((_RB, _HD), lambda i: (i, 0)),
            pl.BlockSpec((_RB, _HD), lambda i: (i, 0)),
        ],
        out_shape=[
            jax.ShapeDtypeStruct((_NN, _D), jnp.float32),
            jax.ShapeDtypeStruct((_NN, _HD), jnp.float32),
            jax.ShapeDtypeStruct((_NN, _HD), jnp.float32),
        ],
    )(h, sa, sb, ua, ub, d, m1, m2, b2)


def _pool_body(h_ref, b_ref, fcw_ref, fcb_ref, o_ref, sums, counts):
    i = pl.program_id(0)

    @pl.when(i == 0)
    def _():
        sums[...] = jnp.zeros_like(sums)
        counts[...] = jnp.zeros_like(counts)

    seg = b_ref[...]
    gid = lax.broadcasted_iota(jnp.int32, (_RB, _G), 1)
    sel = (seg == gid).astype(jnp.float32)
    sums[...] += lax.dot_general(sel, h_ref[...], (((0,), (0,)), ((), ())),
                                 preferred_element_type=jnp.float32,
                                 precision=lax.Precision.HIGHEST)
    counts[...] += lax.dot_general(sel, jnp.ones((_RB, 1), jnp.float32),
                                   (((0,), (0,)), ((), ())),
                                   preferred_element_type=jnp.float32,
                                   precision=lax.Precision.HIGHEST)

    @pl.when(i == pl.num_programs(0) - 1)
    def _():
        pooled = sums[...] / jnp.maximum(counts[...], 1.0)
        o_ref[...] = _mm(pooled, fcw_ref[...]) + fcb_ref[...]


def _pool(h, batch_p, fcw, fcb):
    return pl.pallas_call(
        _pool_body,
        grid=(_GRID,),
        in_specs=[
            pl.BlockSpec((_RB, _D), lambda i: (i, 0)),
            pl.BlockSpec((_RB, 1), lambda i: (i, 0)),
            pl.BlockSpec((_D, _C), lambda i: (0, 0)),
            pl.BlockSpec((1, _C), lambda i: (0, 0)),
        ],
        out_specs=pl.BlockSpec((_G, _C), lambda i: (0, 0)),
        out_shape=jax.ShapeDtypeStruct((_G, _C), jnp.float32),
        scratch_shapes=[
            pltpu.VMEM((_G, _D), jnp.float32),
            pltpu.VMEM((_G, 1), jnp.float32),
        ],
    )(h, batch_p, fcw, fcb)


def kernel(x, edge_index, batch, W, bias, gcn_weight, fc_w, fc_b):
    f32 = jnp.float32
    npad = _NN - _N
    x_p = jnp.pad(x, ((0, npad), (0, 0)))
    epad = _EP - _E
    pad_idx = _N + (jnp.arange(epad, dtype=jnp.int32) % 64)
    src_flat = jnp.concatenate([edge_index[0], pad_idx])
    dst_flat = jnp.concatenate([edge_index[1], pad_idx])
    src64 = src_flat.reshape(_NPART, _CPP, _CHUNK)
    dst64 = dst_flat.reshape(_NPART, _CPP, _CHUNK)
    dst32 = dst_flat.reshape(32, 40, 128)
    batch_p = jnp.pad(batch, (0, npad), constant_values=_G).reshape(_NN, 1)

    m1 = (W - W.T - _GAMMA * jnp.eye(_D, dtype=f32)).T
    m2 = gcn_weight.T
    fcw = fc_w.T
    fcb = fc_b.reshape(1, _C)
    b2 = bias.reshape(1, _D)
    zo = jnp.stack([jnp.zeros((128, 16), f32), jnp.ones((128, 16), f32)])
    zrow = jnp.zeros((_CHUNK, _HD), f32)

    h0, h1 = _hist_call(dst32, zo)
    d, ua, ub = _tc0(x_p, m2, h0, h1)

    h = x_p
    for _ in range(_ITERS):
        sa, sb = _edges_call(ua, ub, src64, dst64, zrow)
        h, ua, ub = _tci(h, sa, sb, ua, ub, d, m1, m2, b2)

    return _pool(h, batch_p, fcw, fcb)
